# Initial kernel scaffold; baseline (speedup 1.0000x reference)
import jax
import jax.numpy as jnp
from jax.experimental import pallas as pl


def kernel(scores, oya, dora, honba_riichi_sticks, action, mask, action_table, info_W, info_b, ln_g, ln_b, scores_W, scores_b, oya_table, dora_table, hrs_W, hrs_b):
    raise NotImplementedError("write your pallas kernel here")


# speedup vs baseline: 1.8984x; 1.8984x over previous
import functools

import jax
import jax.numpy as jnp
from jax import lax
from jax.experimental import pallas as pl
from jax.experimental.pallas import tpu as pltpu
from jax.experimental.pallas import tpu_sc as plsc

B = 4096
S = 50
D = 512
NTAB = 225
TAB_PAD = 256
SENTINEL = 224

BLK = 256
NW = 32
TOT = B * S
PER_W = TOT // NW
CHUNK = 64
NCHUNK = PER_W // CHUNK
IDX_MINOR = 64


def _tc_body(tab_ref, sc_ref, oy_ref, d0, d1, d2, d3, d4, hr_ref,
             lng, lnb, wst, sb, oyat, dtab, hwt, hb, wt, ib, out_ref):
    i = pl.program_id(0)

    @pl.when(i == 0)
    def _():
        out_ref[...] = tab_ref[...]

    @pl.when(i > 0)
    def _():
        x = sc_ref[...]
        mu = jnp.mean(x, axis=-1, keepdims=True)
        xc = x - mu
        var = jnp.mean(xc * xc, axis=-1, keepdims=True)
        xn = xc * lax.rsqrt(var + 1e-5) * lng[...] + lnb[...]
        s_emb = jnp.dot(xn, wst[...], preferred_element_type=jnp.float32) + sb[...]

        oh = (oy_ref[...] == lax.broadcasted_iota(jnp.int32, (BLK, 4), 1))
        oya_emb = jnp.dot(oh.astype(jnp.float32), oyat[...],
                          preferred_element_type=jnp.float32)

        h_emb = jnp.dot(hr_ref[...], hwt[...],
                        preferred_element_type=jnp.float32) + hb[...]

        acc = jnp.dot(s_emb, wt[0:32, :], preferred_element_type=jnp.float32)
        acc += jnp.dot(oya_emb, wt[32:48, :], preferred_element_type=jnp.float32)
        for j, dref in enumerate((d0, d1, d2, d3, d4)):
            ohd = (dref[...] == lax.broadcasted_iota(jnp.int32, (BLK, 38), 1))
            dora_emb = jnp.dot(ohd.astype(jnp.float32), dtab[...],
                               preferred_element_type=jnp.float32)
            lo = 48 + 64 * j
            acc += jnp.dot(dora_emb, wt[lo:lo + 64, :],
                           preferred_element_type=jnp.float32)
        acc += jnp.dot(h_emb, wt[368:384, :], preferred_element_type=jnp.float32)
        out_ref[...] = acc + ib[...]


def _build_combined(tab_pad, scores, oya1, dsplit, hrs, ln_g, ln_b,
                    wst, sb, oyat, dtab, hwt, hb, wt, ib):
    nb = B // BLK
    full = lambda i: (0, 0)
    batch = lambda i: (jnp.maximum(i - 1, 0), 0)
    return pl.pallas_call(
        _tc_body,
        grid=(nb + 1,),
        in_specs=[
            pl.BlockSpec((TAB_PAD, D), full),
            pl.BlockSpec((BLK, 4), batch),
            pl.BlockSpec((BLK, 1), batch),
            pl.BlockSpec((BLK, 1), batch),
            pl.BlockSpec((BLK, 1), batch),
            pl.BlockSpec((BLK, 1), batch),
            pl.BlockSpec((BLK, 1), batch),
            pl.BlockSpec((BLK, 1), batch),
            pl.BlockSpec((BLK, 2), batch),
            pl.BlockSpec((1, 4), full),
            pl.BlockSpec((1, 4), full),
            pl.BlockSpec((4, 32), full),
            pl.BlockSpec((1, 32), full),
            pl.BlockSpec((4, 16), full),
            pl.BlockSpec((38, 64), full),
            pl.BlockSpec((2, 16), full),
            pl.BlockSpec((1, 16), full),
            pl.BlockSpec((384, D), full),
            pl.BlockSpec((1, D), full),
        ],
        out_specs=pl.BlockSpec((BLK, D), lambda i: (i, 0)),
        out_shape=jax.ShapeDtypeStruct((TAB_PAD + B, D), jnp.float32),
    )(tab_pad, scores, oya1, *dsplit, hrs, ln_g, ln_b,
      wst, sb, oyat, dtab, hwt, hb, wt, ib)


def _sc_gather(comb, act2d):
    mesh = plsc.VectorSubcoreMesh(core_axis_name="c", subcore_axis_name="s",
                                  num_cores=2, num_subcores=16)

    @functools.partial(
        pl.kernel,
        out_type=jax.ShapeDtypeStruct((TOT, D), jnp.float32),
        mesh=mesh,
        scratch_types=[
            pltpu.VMEM((NCHUNK, CHUNK), jnp.int32),
            pltpu.VMEM((CHUNK, D), jnp.float32),
            pltpu.VMEM((CHUNK, D), jnp.float32),
            pltpu.SemaphoreType.DMA,
            pltpu.SemaphoreType.DMA,
        ],
    )
    def k(comb_hbm, act_hbm, out_hbm, idx_v, buf_a, buf_b, sem_a, sem_b):
        nc = 2
        wid = lax.axis_index("s") * nc + lax.axis_index("c")
        out0 = wid * PER_W
        pltpu.sync_copy(act_hbm.at[wid], idx_v)

        lane = lax.iota(jnp.int32, 16)

        b0 = wid * (PER_W // S)

        def fix(j, _):
            for kk in range(IDX_MINOR // 16):
                v = idx_v[j, pl.ds(kk * 16, 16)]
                nloc = j * IDX_MINOR + kk * 16 + lane
                b = b0 + ((nloc * 5243) >> 18)
                idx_v[j, pl.ds(kk * 16, 16)] = jnp.where(
                    v == SENTINEL, b + TAB_PAD, v)
            return 0

        lax.fori_loop(0, NCHUNK, fix, 0)

        def gather(c, buf, sem):
            return pltpu.async_copy(comb_hbm.at[idx_v.at[c]], buf, sem)

        def scatter(c, buf):
            pltpu.sync_copy(buf, out_hbm.at[pl.ds(out0 + c * CHUNK, CHUNK)])

        gather(0, buf_a, sem_a)

        def body(p, _):
            c = 2 * p
            gather(c + 1, buf_b, sem_b)
            pltpu.make_async_copy(comb_hbm.at[idx_v.at[c]], buf_a, sem_a).wait()
            scatter(c, buf_a)
            gather(c + 2, buf_a, sem_a)
            pltpu.make_async_copy(comb_hbm.at[idx_v.at[c + 1]], buf_b, sem_b).wait()
            scatter(c + 1, buf_b)
            return 0

        lax.fori_loop(0, NCHUNK // 2 - 1, body, 0)

        c = NCHUNK - 2
        gather(c + 1, buf_b, sem_b)
        pltpu.make_async_copy(comb_hbm.at[idx_v.at[c]], buf_a, sem_a).wait()
        scatter(c, buf_a)
        pltpu.make_async_copy(comb_hbm.at[idx_v.at[c + 1]], buf_b, sem_b).wait()
        scatter(c + 1, buf_b)

    return k(comb, act2d)


def kernel(scores, oya, dora, honba_riichi_sticks, action, mask, action_table,
           info_W, info_b, ln_g, ln_b, scores_W, scores_b, oya_table,
           dora_table, hrs_W, hrs_b):
    del mask
    tab_pad = jnp.zeros((TAB_PAD, D), jnp.float32).at[:NTAB].set(action_table)
    oya1 = oya.astype(jnp.int32).reshape(B, 1)
    dora_i = dora.astype(jnp.int32)
    dsplit = [dora_i[:, j:j + 1] for j in range(5)]
    comb = _build_combined(
        tab_pad, scores, oya1, dsplit, honba_riichi_sticks,
        ln_g.reshape(1, 4), ln_b.reshape(1, 4),
        scores_W.T, scores_b.reshape(1, 32),
        oya_table, dora_table,
        hrs_W.T, hrs_b.reshape(1, 16),
        info_W.T, info_b.reshape(1, D))
    act2d = action.astype(jnp.int32).reshape(NW, NCHUNK, IDX_MINOR)
    out2d = _sc_gather(comb, act2d)
    return out2d.reshape(B, S, D)
